# Initial kernel scaffold; baseline (speedup 1.0000x reference)
import jax
import jax.numpy as jnp
from jax.experimental import pallas as pl


def kernel(pos_u, pos_v, neg_v, u_weight, v_weight):
    raise NotImplementedError("write your pallas kernel here")


# speedup vs baseline: 1.5826x; 1.5826x over previous
import functools

import jax
import jax.numpy as jnp
from jax import lax
from jax.experimental import pallas as pl
from jax.experimental.pallas import tpu as pltpu
from jax.experimental.pallas import tpu_sc as plsc

NC = 2
NS = 16
L = 16
NW = NC * NS


def _sc_scores(pos_u, pos_v, neg_flat, u_w, v_w, B, D, NNEG):
    W = B // NW
    C = 64
    CH = W // C
    G = C // L
    NS_ = 1 + NNEG
    mesh = plsc.VectorSubcoreMesh(
        core_axis_name="c", subcore_axis_name="s",
        num_cores=NC, num_subcores=NS)

    @functools.partial(
        pl.kernel,
        out_type=jax.ShapeDtypeStruct((NW, CH, NS_, C), jnp.float32),
        mesh=mesh,
        compiler_params=pltpu.CompilerParams(
            needs_layout_passes=False, use_tc_tiling_on_sc=False),
        scratch_types=[
            pltpu.VMEM((W,), jnp.int32),
            pltpu.VMEM((W,), jnp.int32),
            pltpu.VMEM((W * NNEG,), jnp.int32),
            pltpu.VMEM((C, D), jnp.float32),
            pltpu.VMEM((C, D), jnp.float32),
            pltpu.VMEM((C * NNEG, D), jnp.float32),
            pltpu.VMEM((NS_, C), jnp.float32),
            pltpu.SemaphoreType.DMA,
        ],
    )
    def body(pos_u_ref, pos_v_ref, neg_ref, u_ref, v_ref, out_ref,
             idx_u, idx_v, idx_n, rows_u, rows_v, rows_n, out_v, sem):
        wid = lax.axis_index("s") * NC + lax.axis_index("c")
        base = wid * W
        pltpu.sync_copy(pos_u_ref.at[pl.ds(base, W)], idx_u)
        pltpu.sync_copy(pos_v_ref.at[pl.ds(base, W)], idx_v)
        pltpu.sync_copy(neg_ref.at[pl.ds(base * NNEG, W * NNEG)], idx_n)

        def chunk(g, carry):
            cu = pltpu.async_copy(u_ref.at[idx_u.at[pl.ds(g * C, C)]], rows_u, sem)
            cv = pltpu.async_copy(v_ref.at[idx_v.at[pl.ds(g * C, C)]], rows_v, sem)
            cn = pltpu.async_copy(
                v_ref.at[idx_n.at[pl.ds(g * C * NNEG, C * NNEG)]], rows_n, sem)
            cu.wait()
            cv.wait()
            cn.wait()
            lanes = lax.iota(jnp.int32, L)
            for t in range(G):
                rowv = lanes + t * L
                nrows = [rowv * NNEG + j for j in range(NNEG)]

                def dstep(d, accs):
                    dv = jnp.full((L,), d, jnp.int32)
                    uc = plsc.load_gather(rows_u, [rowv, dv])
                    vc = plsc.load_gather(rows_v, [rowv, dv])
                    new0 = accs[0] + uc * vc
                    rest = tuple(
                        accs[1 + j] + uc * plsc.load_gather(rows_n, [nrows[j], dv])
                        for j in range(NNEG))
                    return (new0,) + rest

                accs = lax.fori_loop(
                    0, D, dstep,
                    tuple(jnp.zeros((L,), jnp.float32) for _ in range(NS_)))
                for j in range(NS_):
                    out_v[j, pl.ds(t * L, L)] = accs[j]
            pltpu.sync_copy(out_v, out_ref.at[wid, g])
            return carry

        lax.fori_loop(0, CH, chunk, 0)

    return body(pos_u, pos_v, neg_flat, u_w, v_w)


def _tc_loss(scores2d, B, NS_):
    def body(s_ref, o_ref):
        s = s_ref[...]
        r = lax.broadcasted_iota(jnp.int32, s.shape, 0)
        j = r % NS_
        x = jnp.clip(s, -10.0, 10.0)
        z = jnp.where(j == 0, -x, x)
        sp = jnp.maximum(z, 0.0) + jnp.log1p(jnp.exp(-jnp.abs(z)))
        rowsum = jnp.sum(sp, axis=1)
        o_ref[0, 0] = jnp.sum(rowsum) * jnp.float32(1.0 / B)

    out = pl.pallas_call(
        body,
        out_shape=jax.ShapeDtypeStruct((1, 1), jnp.float32),
        out_specs=pl.BlockSpec(memory_space=pltpu.SMEM),
    )(scores2d)
    return out[0, 0]


def kernel(pos_u, pos_v, neg_v, u_weight, v_weight):
    B = pos_u.shape[0]
    NNEG = neg_v.shape[1]
    D = u_weight.shape[1]
    neg_flat = neg_v.astype(jnp.int32).reshape(-1)
    scores = _sc_scores(pos_u.astype(jnp.int32), pos_v.astype(jnp.int32),
                        neg_flat, u_weight, v_weight, B, D, NNEG)
    scores2d = scores.reshape(-1, scores.shape[-1])
    return _tc_loss(scores2d, B, 1 + NNEG)
